# baseline (speedup 1.0000x reference)
import jax
import jax.numpy as jnp
from jax.experimental import pallas as pl


def _moe_step(idx_ref, ew_ref, act_ref, w_ref, bias_ref, resid_ref, out_ref):
    e = pl.program_id(0)
    idx = idx_ref[...]
    ew = ew_ref[...]
    g = jnp.where(idx == e, ew, 0.0)
    g0 = g[:, 0:1]
    g1 = g[:, 1:2]
    act = act_ref[...]
    pe = (g0 * act[:, :64] + g1 * act[:, 64:]).astype(jnp.bfloat16)
    w = w_ref[0].astype(jnp.bfloat16)
    contrib = jax.lax.dot_general(
        pe, w, (((1,), (1,)), ((), ())), preferred_element_type=jnp.float32
    )
    contrib = contrib + (g0 + g1) * bias_ref[0]

    @pl.when(e == 0)
    def _init():
        out_ref[...] = resid_ref[...] + contrib

    @pl.when(e != 0)
    def _acc():
        out_ref[...] += contrib


def kernel(activated, expert_indices, expert_weights, mlp2_weight, mlp2_bias, residual_x):
    B, TOPK, D_FF = activated.shape
    E, D_MODEL, _ = mlp2_weight.shape
    idx = jnp.asarray(expert_indices, jnp.int32)
    act2d = activated.reshape(B, TOPK * D_FF)
    bias3d = mlp2_bias.reshape(E, 1, D_MODEL)

    return pl.pallas_call(
        _moe_step,
        grid=(E,),
        in_specs=[
            pl.BlockSpec((B, TOPK), lambda e: (0, 0)),
            pl.BlockSpec((B, TOPK), lambda e: (0, 0)),
            pl.BlockSpec((B, TOPK * D_FF), lambda e: (0, 0)),
            pl.BlockSpec((1, D_MODEL, D_FF), lambda e: (e, 0, 0)),
            pl.BlockSpec((1, 1, D_MODEL), lambda e: (e, 0, 0)),
            pl.BlockSpec((B, D_MODEL), lambda e: (0, 0)),
        ],
        out_specs=pl.BlockSpec((B, D_MODEL), lambda e: (0, 0)),
        out_shape=jax.ShapeDtypeStruct((B, D_MODEL), jnp.float32),
    )(idx, expert_weights, act2d, mlp2_weight, bias3d, residual_x)


# speedup vs baseline: 1.2636x; 1.2636x over previous
import jax
import jax.numpy as jnp
from jax import lax
from jax.experimental import pallas as pl


def _relayout_step(w_ref, wt_ref):
    d_ff = w_ref.shape[2]
    wt_ref[:, :d_ff] = w_ref[0].astype(jnp.bfloat16)
    wt_ref[:, d_ff:] = w_ref[1].astype(jnp.bfloat16)


def _moe_matmul(idx_ref, ew_ref, act_ref, wt_ref, bias_ref, resid_ref, out_ref):
    B = idx_ref.shape[0]
    E = bias_ref.shape[0]
    D_FF = act_ref.shape[1] // 2
    idx = idx_ref[...]
    ew = ew_ref[...]
    iota_e = lax.broadcasted_iota(jnp.int32, (B, E), 1)
    g0 = jnp.where(iota_e == idx[:, 0:1], ew[:, 0:1], 0.0)
    g1 = jnp.where(iota_e == idx[:, 1:2], ew[:, 1:2], 0.0)
    act = act_ref[...]
    a0 = act[:, :D_FF][:, None, :]
    a1 = act[:, D_FF:][:, None, :]
    p3 = g0[:, :, None] * a0 + g1[:, :, None] * a1
    p = p3.reshape(B, E * D_FF).astype(jnp.bfloat16)
    r = (g0 + g1).astype(jnp.bfloat16)
    acc = lax.dot_general(
        p, wt_ref[...], (((1,), (1,)), ((), ())),
        preferred_element_type=jnp.float32,
    )
    acc += lax.dot_general(
        r, bias_ref[...].astype(jnp.bfloat16), (((1,), (0,)), ((), ())),
        preferred_element_type=jnp.float32,
    )
    out_ref[...] = acc + resid_ref[...]


def kernel(activated, expert_indices, expert_weights, mlp2_weight, mlp2_bias, residual_x):
    B, TOPK, D_FF = activated.shape
    E, D_MODEL, _ = mlp2_weight.shape
    idx = jnp.asarray(expert_indices, jnp.int32)
    act2d = activated.reshape(B, TOPK * D_FF)

    wt = pl.pallas_call(
        _relayout_step,
        grid=(E // 2,),
        in_specs=[pl.BlockSpec((2, D_MODEL, D_FF), lambda e: (e, 0, 0))],
        out_specs=pl.BlockSpec((D_MODEL, 2 * D_FF), lambda e: (0, e)),
        out_shape=jax.ShapeDtypeStruct((D_MODEL, E * D_FF), jnp.bfloat16),
    )(mlp2_weight)

    return pl.pallas_call(
        _moe_matmul,
        in_specs=[
            pl.BlockSpec((B, TOPK), lambda: (0, 0)),
            pl.BlockSpec((B, TOPK), lambda: (0, 0)),
            pl.BlockSpec((B, TOPK * D_FF), lambda: (0, 0)),
            pl.BlockSpec((D_MODEL, E * D_FF), lambda: (0, 0)),
            pl.BlockSpec((E, D_MODEL), lambda: (0, 0)),
            pl.BlockSpec((B, D_MODEL), lambda: (0, 0)),
        ],
        out_specs=pl.BlockSpec((B, D_MODEL), lambda: (0, 0)),
        out_shape=jax.ShapeDtypeStruct((B, D_MODEL), jnp.float32),
    )(idx, expert_weights, act2d, wt, mlp2_bias, residual_x)


# speedup vs baseline: 1.5121x; 1.1966x over previous
import jax
import jax.numpy as jnp
from jax import lax
from jax.experimental import pallas as pl
from jax.experimental.pallas import tpu as pltpu


def _relayout_step(w_ref, wt_ref):
    d_ff = w_ref.shape[2]
    wt_ref[:, :d_ff] = w_ref[0].astype(jnp.bfloat16)
    wt_ref[:, d_ff:] = w_ref[1].astype(jnp.bfloat16)


def _moe_matmul(idx_ref, ew_ref, act_ref, wt_ref, bias_ref, resid_ref, out_ref,
                p_ref, r_ref):
    n = pl.program_id(0)
    B, EK = p_ref.shape
    E = r_ref.shape[1]
    D_FF = EK // E

    @pl.when(n == 0)
    def _build_dispatch():
        idx = idx_ref[...]
        ew = ew_ref[...]
        act = act_ref[...]
        col_e = lax.broadcasted_iota(jnp.int32, (B, EK), 1) // D_FF
        a0t = jnp.tile(act[:, :D_FF], (1, E))
        a1t = jnp.tile(act[:, D_FF:], (1, E))
        m0 = col_e == idx[:, 0:1]
        m1 = col_e == idx[:, 1:2]
        p = (jnp.where(m0, ew[:, 0:1] * a0t, 0.0)
             + jnp.where(m1, ew[:, 1:2] * a1t, 0.0))
        p_ref[...] = p.astype(jnp.bfloat16)
        iota_e = lax.broadcasted_iota(jnp.int32, (B, E), 1)
        g0 = jnp.where(iota_e == idx[:, 0:1], ew[:, 0:1], 0.0)
        g1 = jnp.where(iota_e == idx[:, 1:2], ew[:, 1:2], 0.0)
        r_ref[...] = (g0 + g1).astype(jnp.bfloat16)

    acc = lax.dot_general(
        p_ref[...], wt_ref[...], (((1,), (1,)), ((), ())),
        preferred_element_type=jnp.float32,
    )
    acc += lax.dot_general(
        r_ref[...], bias_ref[...].astype(jnp.bfloat16), (((1,), (0,)), ((), ())),
        preferred_element_type=jnp.float32,
    )
    out_ref[...] = acc + resid_ref[...]


def kernel(activated, expert_indices, expert_weights, mlp2_weight, mlp2_bias, residual_x):
    B, TOPK, D_FF = activated.shape
    E, D_MODEL, _ = mlp2_weight.shape
    idx = jnp.asarray(expert_indices, jnp.int32)
    act2d = activated.reshape(B, TOPK * D_FF)

    wt = pl.pallas_call(
        _relayout_step,
        grid=(E // 2,),
        in_specs=[pl.BlockSpec((2, D_MODEL, D_FF), lambda e: (e, 0, 0))],
        out_specs=pl.BlockSpec((D_MODEL, 2 * D_FF), lambda e: (0, e)),
        out_shape=jax.ShapeDtypeStruct((D_MODEL, E * D_FF), jnp.bfloat16),
    )(mlp2_weight)

    NB = 256
    return pl.pallas_call(
        _moe_matmul,
        grid=(D_MODEL // NB,),
        in_specs=[
            pl.BlockSpec((B, TOPK), lambda n: (0, 0)),
            pl.BlockSpec((B, TOPK), lambda n: (0, 0)),
            pl.BlockSpec((B, TOPK * D_FF), lambda n: (0, 0)),
            pl.BlockSpec((NB, E * D_FF), lambda n: (n, 0)),
            pl.BlockSpec((E, NB), lambda n: (0, n)),
            pl.BlockSpec((B, NB), lambda n: (0, n)),
        ],
        out_specs=pl.BlockSpec((B, NB), lambda n: (0, n)),
        out_shape=jax.ShapeDtypeStruct((B, D_MODEL), jnp.float32),
        scratch_shapes=[
            pltpu.VMEM((B, E * D_FF), jnp.bfloat16),
            pltpu.VMEM((B, E), jnp.bfloat16),
        ],
    )(idx, expert_weights, act2d, wt, mlp2_bias, residual_x)


# speedup vs baseline: 1.8588x; 1.2293x over previous
import jax
import jax.numpy as jnp
from jax import lax
from jax.experimental import pallas as pl
from jax.experimental.pallas import tpu as pltpu


def _relayout_step(w_ref, wt_ref):
    g, _, d_ff = w_ref.shape
    for s in range(g):
        wt_ref[:, s * d_ff:(s + 1) * d_ff] = w_ref[s].astype(jnp.bfloat16)


def _moe_matmul(idx_ref, ew_ref, act_ref, wt_ref, bias_ref, resid_ref, out_ref,
                p_ref, r_ref):
    n = pl.program_id(0)
    B, EK = p_ref.shape
    E = r_ref.shape[1]
    D_FF = EK // E

    @pl.when(n == 0)
    def _build_dispatch():
        idx = idx_ref[...]
        ew = ew_ref[...]
        act = act_ref[...]
        col_e = lax.broadcasted_iota(jnp.int32, (B, EK), 1) // D_FF
        a0t = jnp.tile(act[:, :D_FF], (1, E))
        a1t = jnp.tile(act[:, D_FF:], (1, E))
        m0 = col_e == idx[:, 0:1]
        m1 = col_e == idx[:, 1:2]
        p = (jnp.where(m0, ew[:, 0:1] * a0t, 0.0)
             + jnp.where(m1, ew[:, 1:2] * a1t, 0.0))
        p_ref[...] = p.astype(jnp.bfloat16)
        iota_e = lax.broadcasted_iota(jnp.int32, (B, E), 1)
        g0 = jnp.where(iota_e == idx[:, 0:1], ew[:, 0:1], 0.0)
        g1 = jnp.where(iota_e == idx[:, 1:2], ew[:, 1:2], 0.0)
        r_ref[...] = (g0 + g1).astype(jnp.bfloat16)

    acc = lax.dot_general(
        p_ref[...], wt_ref[...], (((1,), (1,)), ((), ())),
        preferred_element_type=jnp.float32,
    )
    acc += lax.dot_general(
        r_ref[...], bias_ref[...].astype(jnp.bfloat16), (((1,), (0,)), ((), ())),
        preferred_element_type=jnp.float32,
    )
    out_ref[...] = acc + resid_ref[...]


def kernel(activated, expert_indices, expert_weights, mlp2_weight, mlp2_bias, residual_x):
    B, TOPK, D_FF = activated.shape
    E, D_MODEL, _ = mlp2_weight.shape
    idx = jnp.asarray(expert_indices, jnp.int32)
    act2d = activated.reshape(B, TOPK * D_FF)

    GA = 8
    wt = pl.pallas_call(
        _relayout_step,
        grid=(E // GA,),
        in_specs=[pl.BlockSpec((GA, D_MODEL, D_FF), lambda e: (e, 0, 0))],
        out_specs=pl.BlockSpec((D_MODEL, GA * D_FF), lambda e: (0, e)),
        out_shape=jax.ShapeDtypeStruct((D_MODEL, E * D_FF), jnp.bfloat16),
    )(mlp2_weight)

    NB = 256
    return pl.pallas_call(
        _moe_matmul,
        grid=(D_MODEL // NB,),
        in_specs=[
            pl.BlockSpec((B, TOPK), lambda n: (0, 0)),
            pl.BlockSpec((B, TOPK), lambda n: (0, 0)),
            pl.BlockSpec((B, TOPK * D_FF), lambda n: (0, 0)),
            pl.BlockSpec((NB, E * D_FF), lambda n: (n, 0)),
            pl.BlockSpec((E, NB), lambda n: (0, n)),
            pl.BlockSpec((B, NB), lambda n: (0, n)),
        ],
        out_specs=pl.BlockSpec((B, NB), lambda n: (0, n)),
        out_shape=jax.ShapeDtypeStruct((B, D_MODEL), jnp.float32),
        scratch_shapes=[
            pltpu.VMEM((B, E * D_FF), jnp.bfloat16),
            pltpu.VMEM((B, E), jnp.bfloat16),
        ],
    )(idx, expert_weights, act2d, wt, mlp2_bias, residual_x)


# speedup vs baseline: 1.8596x; 1.0004x over previous
import jax
import jax.numpy as jnp
from jax import lax
from jax.experimental import pallas as pl
from jax.experimental.pallas import tpu as pltpu


def _relayout_step(w_ref, wt_ref):
    e, _, d_ff = w_ref.shape
    for s in range(e):
        wt_ref[:, s * d_ff:(s + 1) * d_ff] = w_ref[s].astype(jnp.bfloat16)


def _moe_matmul(idx_ref, ew_ref, act_ref, wt_ref, bias_ref, resid_ref, out_ref,
                p_ref, r_ref):
    n = pl.program_id(0)
    B, EK = p_ref.shape
    E = r_ref.shape[1]
    D_FF = EK // E

    @pl.when(n == 0)
    def _build_dispatch():
        idx = idx_ref[...]
        ew = ew_ref[...]
        act = act_ref[...]
        col_e = lax.broadcasted_iota(jnp.int32, (B, EK), 1) // D_FF
        a0t = jnp.tile(act[:, :D_FF], (1, E))
        a1t = jnp.tile(act[:, D_FF:], (1, E))
        m0 = col_e == idx[:, 0:1]
        m1 = col_e == idx[:, 1:2]
        p = (jnp.where(m0, ew[:, 0:1] * a0t, 0.0)
             + jnp.where(m1, ew[:, 1:2] * a1t, 0.0))
        p_ref[...] = p.astype(jnp.bfloat16)
        iota_e = lax.broadcasted_iota(jnp.int32, (B, E), 1)
        g0 = jnp.where(iota_e == idx[:, 0:1], ew[:, 0:1], 0.0)
        g1 = jnp.where(iota_e == idx[:, 1:2], ew[:, 1:2], 0.0)
        r_ref[...] = (g0 + g1).astype(jnp.bfloat16)

    acc = lax.dot_general(
        p_ref[...], wt_ref[...], (((1,), (1,)), ((), ())),
        preferred_element_type=jnp.float32,
    )
    acc += lax.dot_general(
        r_ref[...], bias_ref[...].astype(jnp.bfloat16), (((1,), (0,)), ((), ())),
        preferred_element_type=jnp.float32,
    )
    out_ref[...] = acc + resid_ref[...]


def kernel(activated, expert_indices, expert_weights, mlp2_weight, mlp2_bias, residual_x):
    B, TOPK, D_FF = activated.shape
    E, D_MODEL, _ = mlp2_weight.shape
    idx = jnp.asarray(expert_indices, jnp.int32)
    act2d = activated.reshape(B, TOPK * D_FF)

    CB = 128
    wt = pl.pallas_call(
        _relayout_step,
        grid=(D_MODEL // CB,),
        in_specs=[pl.BlockSpec((E, CB, D_FF), lambda c: (0, c, 0))],
        out_specs=pl.BlockSpec((CB, E * D_FF), lambda c: (c, 0)),
        out_shape=jax.ShapeDtypeStruct((D_MODEL, E * D_FF), jnp.bfloat16),
    )(mlp2_weight)

    NB = 256
    return pl.pallas_call(
        _moe_matmul,
        grid=(D_MODEL // NB,),
        in_specs=[
            pl.BlockSpec((B, TOPK), lambda n: (0, 0)),
            pl.BlockSpec((B, TOPK), lambda n: (0, 0)),
            pl.BlockSpec((B, TOPK * D_FF), lambda n: (0, 0)),
            pl.BlockSpec((NB, E * D_FF), lambda n: (n, 0)),
            pl.BlockSpec((E, NB), lambda n: (0, n)),
            pl.BlockSpec((B, NB), lambda n: (0, n)),
        ],
        out_specs=pl.BlockSpec((B, NB), lambda n: (0, n)),
        out_shape=jax.ShapeDtypeStruct((B, D_MODEL), jnp.float32),
        scratch_shapes=[
            pltpu.VMEM((B, E * D_FF), jnp.bfloat16),
            pltpu.VMEM((B, E), jnp.bfloat16),
        ],
    )(idx, expert_weights, act2d, wt, mlp2_bias, residual_x)


# speedup vs baseline: 2.1173x; 1.1386x over previous
import jax
import jax.numpy as jnp
from jax import lax
from jax.experimental import pallas as pl
from jax.experimental.pallas import tpu as pltpu


def _moe_fused(idx_ref, ew_ref, act_ref, w_ref, bias_ref, resid_ref, out_ref,
               p_ref, r_ref):
    g = pl.program_id(0)
    NG, B, KB = p_ref.shape
    GE, _, D_FF = w_ref.shape

    @pl.when(g == 0)
    def _build_dispatch():
        idx = idx_ref[...]
        ew = ew_ref[...]
        act = act_ref[...]
        a0t = jnp.tile(act[:, :D_FF], (1, GE))
        a1t = jnp.tile(act[:, D_FF:], (1, GE))
        v0 = ew[:, 0:1] * a0t
        v1 = ew[:, 1:2] * a1t
        colk = lax.broadcasted_iota(jnp.int32, (B, KB), 1) // D_FF
        for gg in range(NG):
            ce = colk + gg * GE
            pgg = (jnp.where(ce == idx[:, 0:1], v0, 0.0)
                   + jnp.where(ce == idx[:, 1:2], v1, 0.0))
            p_ref[gg] = pgg.astype(jnp.bfloat16)
        E = r_ref.shape[1]
        iota_e = lax.broadcasted_iota(jnp.int32, (B, E), 1)
        g0 = jnp.where(iota_e == idx[:, 0:1], ew[:, 0:1], 0.0)
        g1 = jnp.where(iota_e == idx[:, 1:2], ew[:, 1:2], 0.0)
        r_ref[...] = (g0 + g1).astype(jnp.bfloat16)

    wcat = jnp.concatenate(
        [w_ref[s] for s in range(GE)], axis=1).astype(jnp.bfloat16)
    contrib = lax.dot_general(
        p_ref[g], wcat, (((1,), (1,)), ((), ())),
        preferred_element_type=jnp.float32,
    )

    @pl.when(g == 0)
    def _init():
        bias_c = lax.dot_general(
            r_ref[...], bias_ref[...].astype(jnp.bfloat16),
            (((1,), (0,)), ((), ())), preferred_element_type=jnp.float32)
        out_ref[...] = resid_ref[...] + bias_c + contrib

    @pl.when(g != 0)
    def _acc():
        out_ref[...] += contrib


def kernel(activated, expert_indices, expert_weights, mlp2_weight, mlp2_bias, residual_x):
    B, TOPK, D_FF = activated.shape
    E, D_MODEL, _ = mlp2_weight.shape
    idx = jnp.asarray(expert_indices, jnp.int32)
    act2d = activated.reshape(B, TOPK * D_FF)

    GE = 8
    NG = E // GE
    return pl.pallas_call(
        _moe_fused,
        grid=(NG,),
        in_specs=[
            pl.BlockSpec((B, TOPK), lambda g: (0, 0)),
            pl.BlockSpec((B, TOPK), lambda g: (0, 0)),
            pl.BlockSpec((B, TOPK * D_FF), lambda g: (0, 0)),
            pl.BlockSpec((GE, D_MODEL, D_FF), lambda g: (g, 0, 0)),
            pl.BlockSpec((E, D_MODEL), lambda g: (0, 0)),
            pl.BlockSpec((B, D_MODEL), lambda g: (0, 0)),
        ],
        out_specs=pl.BlockSpec((B, D_MODEL), lambda g: (0, 0)),
        out_shape=jax.ShapeDtypeStruct((B, D_MODEL), jnp.float32),
        scratch_shapes=[
            pltpu.VMEM((NG, B, GE * D_FF), jnp.bfloat16),
            pltpu.VMEM((B, E), jnp.bfloat16),
        ],
    )(idx, expert_weights, act2d, mlp2_weight, mlp2_bias, residual_x)


# speedup vs baseline: 2.1915x; 1.0350x over previous
import jax
import jax.numpy as jnp
from jax import lax
from jax.experimental import pallas as pl
from jax.experimental.pallas import tpu as pltpu


def _moe_fused(idx_ref, ew_ref, act_ref, w_ref, bias_ref, resid_ref, out_ref,
               p_ref, r_ref):
    g = pl.program_id(0)
    NG, B, KB = p_ref.shape
    GE, _, D_FF = w_ref.shape

    @pl.when(g == 0)
    def _build_dispatch():
        idx = idx_ref[...]
        ew = ew_ref[...]
        act = act_ref[...]
        a0t = jnp.tile(act[:, :D_FF], (1, GE))
        a1t = jnp.tile(act[:, D_FF:], (1, GE))
        v0 = ew[:, 0:1] * a0t
        v1 = ew[:, 1:2] * a1t
        colk = lax.broadcasted_iota(jnp.int32, (B, KB), 1) // D_FF
        for gg in range(NG):
            ce = colk + gg * GE
            pgg = (jnp.where(ce == idx[:, 0:1], v0, 0.0)
                   + jnp.where(ce == idx[:, 1:2], v1, 0.0))
            p_ref[gg] = pgg.astype(jnp.bfloat16)
        E = r_ref.shape[1]
        iota_e = lax.broadcasted_iota(jnp.int32, (B, E), 1)
        g0 = jnp.where(iota_e == idx[:, 0:1], ew[:, 0:1], 0.0)
        g1 = jnp.where(iota_e == idx[:, 1:2], ew[:, 1:2], 0.0)
        r_ref[...] = (g0 + g1).astype(jnp.bfloat16)

    wcat = jnp.concatenate(
        [w_ref[s] for s in range(GE)], axis=1).astype(jnp.bfloat16)
    contrib = lax.dot_general(
        p_ref[g], wcat, (((1,), (1,)), ((), ())),
        preferred_element_type=jnp.float32,
    )

    @pl.when(g == 0)
    def _init():
        bias_c = lax.dot_general(
            r_ref[...], bias_ref[...].astype(jnp.bfloat16),
            (((1,), (0,)), ((), ())), preferred_element_type=jnp.float32)
        out_ref[...] = resid_ref[...] + bias_c + contrib

    @pl.when(g != 0)
    def _acc():
        out_ref[...] += contrib


def kernel(activated, expert_indices, expert_weights, mlp2_weight, mlp2_bias, residual_x):
    B, TOPK, D_FF = activated.shape
    E, D_MODEL, _ = mlp2_weight.shape
    idx = jnp.asarray(expert_indices, jnp.int32)
    act2d = activated.reshape(B, TOPK * D_FF)

    GE = 16
    NG = E // GE
    return pl.pallas_call(
        _moe_fused,
        grid=(NG,),
        in_specs=[
            pl.BlockSpec((B, TOPK), lambda g: (0, 0)),
            pl.BlockSpec((B, TOPK), lambda g: (0, 0)),
            pl.BlockSpec((B, TOPK * D_FF), lambda g: (0, 0)),
            pl.BlockSpec((GE, D_MODEL, D_FF), lambda g: (g, 0, 0)),
            pl.BlockSpec((E, D_MODEL), lambda g: (0, 0)),
            pl.BlockSpec((B, D_MODEL), lambda g: (0, 0)),
        ],
        out_specs=pl.BlockSpec((B, D_MODEL), lambda g: (0, 0)),
        out_shape=jax.ShapeDtypeStruct((B, D_MODEL), jnp.float32),
        scratch_shapes=[
            pltpu.VMEM((NG, B, GE * D_FF), jnp.bfloat16),
            pltpu.VMEM((B, E), jnp.bfloat16),
        ],
    )(idx, expert_weights, act2d, mlp2_weight, mlp2_bias, residual_x)
